# Initial kernel scaffold; baseline (speedup 1.0000x reference)
import jax
import jax.numpy as jnp
from jax.experimental import pallas as pl


def kernel(coord_v):
    raise NotImplementedError("write your pallas kernel here")


# speedup vs baseline: 5.1009x; 5.1009x over previous
import functools

import jax
import jax.numpy as jnp
import numpy as np
from jax import lax
from jax.experimental import pallas as pl
from jax.experimental.pallas import tpu as pltpu
from jax.experimental.pallas import tpu_sc as plsc

NMC = 8
L = 32
GRID = NMC * L * L
TILE = 2 * GRID
LANES = 16

_EDGES = tuple(np.float32(10.0 ** (j / 4.0)) for j in range(1, NMC))


def _make_sc_call(n_batch, n):
    assert n % LANES == 0
    groups = n // LANES
    n_workers = 32
    assert n_batch % n_workers == 0
    b_per_w = n_batch // n_workers

    mesh = plsc.VectorSubcoreMesh(core_axis_name="c", subcore_axis_name="s")

    @functools.partial(
        pl.kernel,
        mesh=mesh,
        compiler_params=pltpu.CompilerParams(needs_layout_passes=False),
        out_type=jax.ShapeDtypeStruct((n_batch, TILE), jnp.float32),
        scratch_types=[
            pltpu.VMEM((TILE,), jnp.float32),
            pltpu.VMEM((b_per_w, n), jnp.float32),
            pltpu.VMEM((b_per_w, n), jnp.float32),
            pltpu.VMEM((b_per_w, n), jnp.float32),
        ],
    )
    def sc_kernel(xs_hbm, ys_hbm, ms_hbm, out_hbm, buf, xv, yv, mv):
        wid = lax.axis_index("s") * 2 + lax.axis_index("c")
        base_b = wid * b_per_w

        ones_f = jnp.full((LANES,), 1.0, jnp.float32)
        zeros_f = jnp.zeros((LANES,), jnp.float32)

        pltpu.sync_copy(xs_hbm.at[pl.ds(base_b, b_per_w)], xv)
        pltpu.sync_copy(ys_hbm.at[pl.ds(base_b, b_per_w)], yv)
        pltpu.sync_copy(ms_hbm.at[pl.ds(base_b, b_per_w)], mv)

        def fill(i, _):
            buf[pl.ds(i * LANES, LANES)] = ones_f
            buf[pl.ds(GRID + i * LANES, LANES)] = zeros_f
            return _

        lax.fori_loop(0, GRID // LANES, fill, None)

        def per_batch(b, _):
            bases = []
            for g in range(groups):
                x = xv[b, pl.ds(g * LANES, LANES)]
                y = yv[b, pl.ds(g * LANES, LANES)]
                m = mv[b, pl.ds(g * LANES, LANES)]
                xi = (x * np.float32(L)).astype(jnp.int32)
                yi = (y * np.float32(L)).astype(jnp.int32)
                mi = jnp.zeros((LANES,), jnp.int32)
                one_i = jnp.ones((LANES,), jnp.int32)
                for e in _EDGES:
                    mi = mi + jnp.where(m >= e, one_i, 0)
                bases.append(mi * (L * L) + yi * L + xi)
            for base in bases:
                plsc.store_scatter(buf, [base], zeros_f)
                plsc.store_scatter(buf, [base + GRID], ones_f)
            pltpu.sync_copy(buf, out_hbm.at[base_b + b])
            for base in bases:
                plsc.store_scatter(buf, [base], ones_f)
                plsc.store_scatter(buf, [base + GRID], zeros_f)
            return _

        lax.fori_loop(0, b_per_w, per_batch, None)

    return sc_kernel


@jax.jit
def kernel(coord_v):
    n_batch = coord_v.shape[0]
    n = coord_v.shape[1] // 3
    c = coord_v.reshape(n_batch, n, 3)
    xs = c[:, :, 0]
    ys = c[:, :, 1]
    ms = c[:, :, 2]
    out = _make_sc_call(n_batch, n)(xs, ys, ms)
    return out.reshape(n_batch, 2 * NMC, L, L)
